# baseline (speedup 1.0000x reference)
import functools

import jax
import jax.numpy as jnp
from jax import lax
from jax.experimental import pallas as pl
from jax.experimental.pallas import tpu as pltpu
from jax.experimental.pallas import tpu_sc as plsc

_SEQ = 8192
_DIM = 1024

_info = plsc.get_sparse_core_info()
_NC, _NS = _info.num_cores, _info.num_subcores
_HALF = _NS // 2

_SC_ROWS = _SEQ // _NC
_DOM = _SC_ROWS // _HALF
_CH = 64
_NROUND = _DOM // _CH

_mesh = plsc.VectorSubcoreMesh(core_axis_name="c", subcore_axis_name="s")


@functools.partial(
    pl.kernel,
    mesh=_mesh,
    out_type=jax.ShapeDtypeStruct((_SEQ, _DIM), jnp.float32),
    scratch_types=[
        pltpu.VMEM_SHARED((2, _HALF, _CH, _DIM), jnp.float32),
        pltpu.SemaphoreType.DMA,
    ],
)
def _pos_emb_copy(table_hbm, out_hbm, spbuf, sem):
    c = lax.axis_index("c")
    s = lax.axis_index("s")
    sc_base = c * _SC_ROWS
    is_reader = s < _HALF
    t = jnp.where(is_reader, s, s - _HALF)
    dom_base = sc_base + t * _DOM

    for r in range(_NROUND + 1):
        if r < _NROUND:
            @pl.when(is_reader)
            def _():
                pltpu.async_copy(
                    table_hbm.at[pl.ds(dom_base + r * _CH, _CH)],
                    spbuf.at[r % 2, t],
                    sem,
                ).wait()
        if r >= 1:
            @pl.when(jnp.logical_not(is_reader))
            def _():
                rr = r - 1
                pltpu.async_copy(
                    spbuf.at[rr % 2, t],
                    out_hbm.at[pl.ds(dom_base + rr * _CH, _CH)],
                    sem,
                ).wait()
        plsc.subcore_barrier()


def kernel(hidden_embs, position_embedding_table):
    del hidden_embs
    return _pos_emb_copy(position_embedding_table)


# speedup vs baseline: 1.1681x; 1.1681x over previous
import functools

import jax
import jax.numpy as jnp
from jax import lax
from jax.experimental import pallas as pl
from jax.experimental.pallas import tpu as pltpu
from jax.experimental.pallas import tpu_sc as plsc

_SEQ = 8192
_DIM = 1024

_info = plsc.get_sparse_core_info()
_NC, _NS = _info.num_cores, _info.num_subcores
_NW = _NC * _NS
_ROWS_PER_W = _SEQ // _NW

_CH = 32
_NCHUNK = _ROWS_PER_W // _CH
_NB = 3

_mesh = plsc.VectorSubcoreMesh(core_axis_name="c", subcore_axis_name="s")


@functools.partial(
    pl.kernel,
    mesh=_mesh,
    out_type=jax.ShapeDtypeStruct((_SEQ, _DIM), jnp.float32),
    scratch_types=(
        [pltpu.VMEM((_CH, _DIM), jnp.float32) for _ in range(_NB)]
        + [pltpu.SemaphoreType.DMA for _ in range(_NB)]
        + [pltpu.SemaphoreType.DMA for _ in range(_NB)]
    ),
)
def _pos_emb_copy(table_hbm, out_hbm, *scratch):
    bufs = scratch[:_NB]
    rsems = scratch[_NB:2 * _NB]
    wsems = scratch[2 * _NB:]

    wid = lax.axis_index("s") * _NC + lax.axis_index("c")
    base = wid * _ROWS_PER_W

    reads = [None] * _NCHUNK
    writes = [None] * _NCHUNK

    for i in range(min(_NB, _NCHUNK)):
        reads[i] = pltpu.async_copy(
            table_hbm.at[pl.ds(base + i * _CH, _CH)], bufs[i], rsems[i]
        )
    for i in range(_NCHUNK):
        b = i % _NB
        reads[i].wait()
        writes[i] = pltpu.async_copy(
            bufs[b], out_hbm.at[pl.ds(base + i * _CH, _CH)], wsems[b]
        )
        j = i + _NB
        if j < _NCHUNK:
            writes[i].wait()
            reads[j] = pltpu.async_copy(
                table_hbm.at[pl.ds(base + j * _CH, _CH)], bufs[b], rsems[b]
            )
    for i in range(max(0, _NCHUNK - _NB), _NCHUNK):
        writes[i].wait()


def kernel(hidden_embs, position_embedding_table):
    del hidden_embs
    return _pos_emb_copy(position_embedding_table)
